# baseline (speedup 1.0000x reference)
import jax
import jax.numpy as jnp
from jax.experimental import pallas as pl

K_TAPS = 4
DILATION = 2
BC = 96


def kernel(x):
    B, C, T = x.shape

    def body(x_ref, o_ref):
        xv = x_ref[0]
        for k in range(K_TAPS):
            s = (K_TAPS - 1 - k) * DILATION
            if s == 0:
                o_ref[0, k] = xv
            else:
                o_ref[0, k] = jnp.concatenate(
                    [jnp.zeros((BC, s), jnp.float32), xv[:, : T - s]], axis=1
                )

    out4 = pl.pallas_call(
        body,
        grid=(B, C // BC),
        in_specs=[pl.BlockSpec((1, BC, T), lambda b, c: (b, c, 0))],
        out_specs=pl.BlockSpec((1, K_TAPS, BC, T), lambda b, c: (b, 0, c, 0)),
        out_shape=jax.ShapeDtypeStruct((B, K_TAPS, C, T), jnp.float32),
    )(x)
    return out4.reshape(B, K_TAPS * C, T)


# speedup vs baseline: 1.0474x; 1.0474x over previous
import jax
import jax.numpy as jnp
from jax.experimental import pallas as pl

K_TAPS = 4
DILATION = 2
BC = 192


def kernel(x):
    B, C, T = x.shape

    def body(x_ref, o_ref):
        xv = x_ref[0]
        for k in range(K_TAPS):
            s = (K_TAPS - 1 - k) * DILATION
            if s == 0:
                o_ref[0, k] = xv
            else:
                o_ref[0, k] = jnp.concatenate(
                    [jnp.zeros((BC, s), jnp.float32), xv[:, : T - s]], axis=1
                )

    out4 = pl.pallas_call(
        body,
        grid=(B, C // BC),
        in_specs=[pl.BlockSpec((1, BC, T), lambda b, c: (b, c, 0))],
        out_specs=pl.BlockSpec((1, K_TAPS, BC, T), lambda b, c: (b, 0, c, 0)),
        out_shape=jax.ShapeDtypeStruct((B, K_TAPS, C, T), jnp.float32),
    )(x)
    return out4.reshape(B, K_TAPS * C, T)
